# baseline (speedup 1.0000x reference)
import functools

import jax
import jax.numpy as jnp
from jax import lax
from jax.experimental import pallas as pl
from jax.experimental.pallas import tpu as pltpu
from jax.experimental.pallas import tpu_sc as plsc

_NC, _NS, _LANES = 2, 16, 16
_NW = _NC * _NS
_CHUNK = 128


def _dot(a, b):
    return jnp.dot(a, b, preferred_element_type=jnp.float32)


def _proj2_body(x_ref, w1_ref, w2_ref, o1_ref, o2_ref):
    x = x_ref[...]
    o1_ref[...] = _dot(x, w1_ref[...])
    o2_ref[...] = _dot(x, w2_ref[...])


def _proj2(x, w1, w2, blk):
    n, d = x.shape
    return pl.pallas_call(
        _proj2_body,
        grid=(n // blk,),
        in_specs=[pl.BlockSpec((blk, d), lambda i: (i, 0)),
                  pl.BlockSpec(w1.shape, lambda i: (0, 0)),
                  pl.BlockSpec(w2.shape, lambda i: (0, 0))],
        out_specs=[pl.BlockSpec((blk, w1.shape[1]), lambda i: (i, 0)),
                   pl.BlockSpec((blk, w2.shape[1]), lambda i: (i, 0))],
        out_shape=[jax.ShapeDtypeStruct((n, w1.shape[1]), jnp.float32),
                   jax.ShapeDtypeStruct((n, w2.shape[1]), jnp.float32)],
    )(x, w1, w2)


def _dot_t(xt, w):
    return jax.lax.dot_general(xt, w, (((0,), (0,)), ((), ())),
                               preferred_element_type=jnp.float32)


def _proj2t_body(xt_ref, w1_ref, w2_ref, o1_ref, o2_ref):
    xt = xt_ref[...]
    o1_ref[...] = _dot_t(xt, w1_ref[...])
    o2_ref[...] = _dot_t(xt, w2_ref[...])


def _proj2t(xt, w1, w2, blk):
    d, n = xt.shape
    return pl.pallas_call(
        _proj2t_body,
        grid=(n // blk,),
        in_specs=[pl.BlockSpec((d, blk), lambda i: (0, i)),
                  pl.BlockSpec(w1.shape, lambda i: (0, 0)),
                  pl.BlockSpec(w2.shape, lambda i: (0, 0))],
        out_specs=[pl.BlockSpec((blk, w1.shape[1]), lambda i: (i, 0)),
                   pl.BlockSpec((blk, w2.shape[1]), lambda i: (i, 0))],
        out_shape=[jax.ShapeDtypeStruct((n, w1.shape[1]), jnp.float32),
                   jax.ShapeDtypeStruct((n, w2.shape[1]), jnp.float32)],
    )(xt, w1, w2)


def _proj1t_body(xt_ref, w_ref, o_ref):
    o_ref[...] = _dot_t(xt_ref[...], w_ref[...])


def _proj1t(xt, w, blk):
    d, n = xt.shape
    return pl.pallas_call(
        _proj1t_body,
        grid=(n // blk,),
        in_specs=[pl.BlockSpec((d, blk), lambda i: (0, i)),
                  pl.BlockSpec(w.shape, lambda i: (0, 0))],
        out_specs=pl.BlockSpec((blk, w.shape[1]), lambda i: (i, 0)),
        out_shape=jax.ShapeDtypeStruct((n, w.shape[1]), jnp.float32),
    )(xt, w)


def _sigmoid(x):
    return 0.5 * jnp.tanh(0.5 * x) + 0.5


def _act_body(n_rows, half, residual, *refs):
    if residual:
        xg_ref, sums_ref, sqs_ref, gb_ref, aft_ref, o_ref = refs
    else:
        xg_ref, sums_ref, sqs_ref, gb_ref, o_ref = refs
    cg = xg_ref[...]
    s = jnp.sum(sums_ref[...], axis=0, keepdims=True)
    q = jnp.sum(sqs_ref[...], axis=0, keepdims=True)
    inv_n = 1.0 / float(n_rows)
    mu = s * inv_n
    var = q * inv_n - mu * mu
    inv = lax.rsqrt(var + 1e-5)
    y = (cg - mu) * (inv * gb_ref[0:1, :]) + gb_ref[1:2, :]
    core = y[:, :half]
    gate = y[:, half:]
    x = core * _sigmoid(core) * _sigmoid(gate)
    if residual:
        o_ref[...] = jnp.transpose(x, (1, 0)) + aft_ref[...]
    else:
        o_ref[...] = x


def _act(xg, partials, gb, half, residual, blk, aft=None):
    n, wc = xg.shape
    ps = partials.reshape(_NW, 2, wc)
    sums = ps[:, 0, :]
    sqs = ps[:, 1, :]
    body = functools.partial(_act_body, n, half, residual)
    in_specs = [pl.BlockSpec((blk, wc), lambda i: (i, 0)),
                pl.BlockSpec((_NW, wc), lambda i: (0, 0)),
                pl.BlockSpec((_NW, wc), lambda i: (0, 0)),
                pl.BlockSpec((2, wc), lambda i: (0, 0))]
    args = [xg, sums, sqs, gb]
    if residual:
        in_specs.append(pl.BlockSpec((half, blk), lambda i: (0, i)))
        args.append(aft)
        out_specs = pl.BlockSpec((half, blk), lambda i: (0, i))
        out_shape = jax.ShapeDtypeStruct((half, n), jnp.float32)
    else:
        out_specs = pl.BlockSpec((blk, half), lambda i: (i, 0))
        out_shape = jax.ShapeDtypeStruct((n, half), jnp.float32)
    return pl.pallas_call(
        body,
        grid=(n // blk,),
        in_specs=in_specs,
        out_specs=out_specs,
        out_shape=out_shape,
    )(*args)


def _nodeproj_body(a0_ref, a1_ref, vf_ref, wo_ref, wj_ref, o_ref):
    nv = _dot(a0_ref[...] + a1_ref[...], wo_ref[...]) + vf_ref[...]
    o_ref[...] = _dot(nv, wj_ref[...])


def _nodeproj(a0, a1, vf, wo, wj, blk):
    n, d = vf.shape
    return pl.pallas_call(
        _nodeproj_body,
        grid=(n // blk,),
        in_specs=[pl.BlockSpec((blk, d), lambda i: (i, 0)),
                  pl.BlockSpec((blk, d), lambda i: (i, 0)),
                  pl.BlockSpec((blk, d), lambda i: (i, 0)),
                  pl.BlockSpec((d, d), lambda i: (0, 0)),
                  pl.BlockSpec((d, wj.shape[1]), lambda i: (0, 0))],
        out_specs=pl.BlockSpec((blk, wj.shape[1]), lambda i: (i, 0)),
        out_shape=jax.ShapeDtypeStruct((n, wj.shape[1]), jnp.float32),
    )(a0, a1, vf, wo, wj)


def _gsum_sc(tables, idxs, dense):
    nt = len(tables)
    m = idxs[0].shape[0]
    width = tables[0].shape[1]
    chunk = 64
    nseg = width // _LANES
    nchunk = m // chunk
    iters = (nchunk + _NW - 1) // _NW
    mesh = plsc.VectorSubcoreMesh(core_axis_name="c", subcore_axis_name="s")
    scratch = ([pltpu.VMEM((chunk,), jnp.int32) for _ in range(2 * nt)]
               + [pltpu.VMEM((chunk, width), jnp.float32)
                  for _ in range(2 * (nt + 1))]
               + [pltpu.VMEM((2, width), jnp.float32)]
               + [pltpu.SemaphoreType.DMA for _ in range(4)])

    def body(*refs):
        table_refs = refs[:nt]
        idx_refs = refs[nt:2 * nt]
        dense_ref = refs[2 * nt]
        out_ref = refs[2 * nt + 1]
        pstat_ref = refs[2 * nt + 2]
        sc = refs[2 * nt + 3:]
        ib = [sc[p * nt:(p + 1) * nt] for p in range(2)]
        bf = [sc[2 * nt + p * (nt + 1):2 * nt + (p + 1) * (nt + 1)]
              for p in range(2)]
        sacc = sc[4 * nt + 2]
        isem = sc[4 * nt + 3:4 * nt + 5]
        gsem = sc[4 * nt + 5:4 * nt + 7]
        cid = lax.axis_index("c")
        sid = lax.axis_index("s")
        wid = sid * _NC + cid
        zero = jnp.zeros((_LANES,), jnp.float32)

        for g in range(nseg):
            sacc[0, pl.ds(g * _LANES, _LANES)] = zero
            sacc[1, pl.ds(g * _LANES, _LANES)] = zero

        def load_idx(i, p):
            @pl.when(i * _NW + wid < nchunk)
            def _():
                base = (i * _NW + wid) * chunk
                for ti in range(nt):
                    pltpu.async_copy(idx_refs[ti].at[pl.ds(base, chunk)],
                                     ib[p][ti], isem[p])

        def wait_idx(i, p):
            @pl.when(i * _NW + wid < nchunk)
            def _():
                base = (i * _NW + wid) * chunk
                for ti in range(nt):
                    pltpu.make_async_copy(
                        idx_refs[ti].at[pl.ds(base, chunk)],
                        ib[p][ti], isem[p]).wait()

        def issue_gathers(i, p):
            @pl.when(i * _NW + wid < nchunk)
            def _():
                base = (i * _NW + wid) * chunk
                for ti in range(nt):
                    pltpu.async_copy(table_refs[ti].at[ib[p][ti]],
                                     bf[p][ti], gsem[p])
                pltpu.async_copy(dense_ref.at[pl.ds(base, chunk)],
                                 bf[p][nt], gsem[p])

        def wait_gathers(i, p):
            @pl.when(i * _NW + wid < nchunk)
            def _():
                base = (i * _NW + wid) * chunk
                for ti in range(nt):
                    pltpu.make_async_copy(table_refs[ti].at[ib[p][ti]],
                                          bf[p][ti], gsem[p]).wait()
                pltpu.make_async_copy(dense_ref.at[pl.ds(base, chunk)],
                                      bf[p][nt], gsem[p]).wait()

        def add_and_write(i, p):
            @pl.when(i * _NW + wid < nchunk)
            def _():
                def add_row(r, carry):
                    accs = list(carry)
                    for g in range(nseg):
                        sl = pl.ds(g * _LANES, _LANES)
                        x = bf[p][0][r, sl]
                        for ti in range(1, nt + 1):
                            x = x + bf[p][ti][r, sl]
                        bf[p][0][r, sl] = x
                        accs[2 * g] = accs[2 * g] + x
                        accs[2 * g + 1] = accs[2 * g + 1] + x * x
                    return tuple(accs)

                accs = lax.fori_loop(0, chunk, add_row,
                                     tuple(zero for _ in range(2 * nseg)))
                for g in range(nseg):
                    sl = pl.ds(g * _LANES, _LANES)
                    sacc[0, sl] = sacc[0, sl] + accs[2 * g]
                    sacc[1, sl] = sacc[1, sl] + accs[2 * g + 1]
                pltpu.sync_copy(
                    bf[p][0],
                    out_ref.at[pl.ds((i * _NW + wid) * chunk, chunk)])

        load_idx(0, 0)
        wait_idx(0, 0)
        issue_gathers(0, 0)

        def pair(ih, carry):
            for p in range(2):
                i = ih * 2 + p
                q = 1 - p
                load_idx(i + 1, q)
                wait_gathers(i, p)
                wait_idx(i + 1, q)
                issue_gathers(i + 1, q)
                add_and_write(i, p)
            return carry

        lax.fori_loop(0, (iters + 1) // 2, pair, 0)
        pltpu.sync_copy(sacc, pstat_ref.at[cid, sid])

    fn = pl.kernel(
        body,
        out_type=(jax.ShapeDtypeStruct((m, width), jnp.float32),
                  jax.ShapeDtypeStruct((_NC, _NS, 2, width), jnp.float32)),
        mesh=mesh,
        scratch_types=scratch,
        compiler_params=pltpu.CompilerParams(use_tc_tiling_on_sc=True),
    )
    return fn(*tables, *idxs, dense)


def _scatter_sc(msg, src, n):
    e, d = msg.shape
    nseg = d // _LANES
    nchunk = e // _CHUNK
    per_core = nchunk // _NC
    iters = (per_core + _NS - 1) // _NS
    npieces = (n + _CHUNK - 1) // _CHUNK
    plast = n - (npieces - 1) * _CHUNK
    piters = (npieces + _NS - 1) // _NS
    mesh = plsc.VectorSubcoreMesh(core_axis_name="c", subcore_axis_name="s")
    scratch = [pltpu.VMEM((_CHUNK, d), jnp.float32),
               pltpu.VMEM((_CHUNK, d), jnp.float32),
               pltpu.VMEM((_CHUNK,), jnp.int32),
               pltpu.VMEM((_CHUNK,), jnp.int32),
               pltpu.SemaphoreType.DMA,
               pltpu.SemaphoreType.DMA,
               pltpu.VMEM_SHARED((n, d), jnp.float32)]

    def body(msg_hbm, src_hbm, out_hbm, mbuf0, mbuf1, ibuf0, ibuf1,
             lsem0, lsem1, acc):
        mbuf = (mbuf0, mbuf1)
        ibuf = (ibuf0, ibuf1)
        lsem = (lsem0, lsem1)
        c = lax.axis_index("c")
        s = lax.axis_index("s")
        zero = jnp.zeros((_LANES,), jnp.float32)

        def zrow(r, cy):
            for g in range(nseg):
                mbuf0[r, pl.ds(g * _LANES, _LANES)] = zero
            return cy

        lax.fori_loop(0, _CHUNK, zrow, 0)

        def zpiece(i, cy):
            p = i * _NS + s

            @pl.when(p < npieces - 1)
            def _():
                pltpu.sync_copy(mbuf0, acc.at[pl.ds(p * _CHUNK, _CHUNK)])

            @pl.when(p == npieces - 1)
            def _():
                pltpu.sync_copy(mbuf0.at[pl.ds(0, plast)],
                                acc.at[pl.ds(p * _CHUNK, plast)])

            return cy

        lax.fori_loop(0, piters, zpiece, 0)
        plsc.subcore_barrier()

        def load(i, p):
            @pl.when(i * _NS + s < per_core)
            def _():
                base = (c * per_core + i * _NS + s) * _CHUNK
                pltpu.async_copy(src_hbm.at[pl.ds(base, _CHUNK)],
                                 ibuf[p], lsem[p])
                pltpu.async_copy(msg_hbm.at[pl.ds(base, _CHUNK)],
                                 mbuf[p], lsem[p])

        def wait_load(i, p):
            @pl.when(i * _NS + s < per_core)
            def _():
                base = (c * per_core + i * _NS + s) * _CHUNK
                pltpu.make_async_copy(src_hbm.at[pl.ds(base, _CHUNK)],
                                      ibuf[p], lsem[p]).wait()
                pltpu.make_async_copy(msg_hbm.at[pl.ds(base, _CHUNK)],
                                      mbuf[p], lsem[p]).wait()

        def scat(i, p):
            @pl.when(i * _NS + s < per_core)
            def _():
                pltpu.sync_copy(mbuf[p], acc.at[ibuf[p]], add=True)

        load(0, 0)

        def pair(ih, cy):
            for p in range(2):
                i = ih * 2 + p
                load(i + 1, 1 - p)
                wait_load(i, p)
                scat(i, p)
            return cy

        lax.fori_loop(0, (iters + 1) // 2, pair, 0)
        plsc.subcore_barrier()

        def wpiece(i, cy):
            p = i * _NS + s

            @pl.when(p < npieces - 1)
            def _():
                pltpu.sync_copy(acc.at[pl.ds(p * _CHUNK, _CHUNK)],
                                out_hbm.at[c, pl.ds(p * _CHUNK, _CHUNK)])

            @pl.when(p == npieces - 1)
            def _():
                pltpu.sync_copy(acc.at[pl.ds(p * _CHUNK, plast)],
                                out_hbm.at[c, pl.ds(p * _CHUNK, plast)])

            return cy

        lax.fori_loop(0, piters, wpiece, 0)

    fn = pl.kernel(
        body,
        out_type=jax.ShapeDtypeStruct((_NC, n, d), jnp.float32),
        mesh=mesh,
        scratch_types=scratch,
        compiler_params=pltpu.CompilerParams(use_tc_tiling_on_sc=True),
    )
    return fn(msg, src)


def kernel(vertex_feat, edge_feat, angle_feat, atom_core_w, atom_gate_w,
           atom_bn_core_g, atom_bn_core_b, atom_bn_gate_g, atom_bn_gate_b,
           atom_out_w, angle_core_w, angle_gate_w, angle_bn_core_g,
           angle_bn_core_b, angle_bn_gate_g, angle_bn_gate_b,
           edge_index, k_idx, j_idx, i_idx):
    n, ad = vertex_feat.shape
    e, bd = edge_feat.shape
    t, gd = angle_feat.shape

    src = edge_index[0].astype(jnp.int32)
    dst = edge_index[1].astype(jnp.int32)
    kk = k_idx.astype(jnp.int32)
    jj = j_idx.astype(jnp.int32)
    ii = i_idx.astype(jnp.int32)

    w_src = jnp.concatenate([atom_core_w[:, :ad].T, atom_gate_w[:, :ad].T], 1)
    w_dst = jnp.concatenate([atom_core_w[:, ad + bd:].T,
                             atom_gate_w[:, ad + bd:].T], 1)
    w_e = jnp.concatenate([atom_core_w[:, ad:ad + bd].T,
                           atom_gate_w[:, ad:ad + bd].T], 1)
    gb_atom = jnp.stack([jnp.concatenate([atom_bn_core_g, atom_bn_gate_g]),
                         jnp.concatenate([atom_bn_core_b, atom_bn_gate_b])])
    w_j = jnp.concatenate([angle_core_w[:, :ad].T, angle_gate_w[:, :ad].T], 1)
    w_k = jnp.concatenate([angle_core_w[:, ad:ad + bd].T,
                           angle_gate_w[:, ad:ad + bd].T], 1)
    w_i = jnp.concatenate([angle_core_w[:, ad + bd:ad + 2 * bd].T,
                           angle_gate_w[:, ad + bd:ad + 2 * bd].T], 1)
    w_a = jnp.concatenate([angle_core_w[:, ad + 2 * bd:].T,
                           angle_gate_w[:, ad + 2 * bd:].T], 1)
    gb_ang = jnp.stack([jnp.concatenate([angle_bn_core_g, angle_bn_gate_g]),
                        jnp.concatenate([angle_bn_core_b, angle_bn_gate_b])])

    ef_t = jnp.transpose(edge_feat, (1, 0))
    af_t = jnp.transpose(angle_feat, (1, 0))

    t_src, t_dst = _proj2(vertex_feat, w_src, w_dst, blk=1000)
    pe = _proj1t(ef_t, w_e, blk=1280)
    cg_g, pstat_a = _gsum_sc([t_src, t_dst], [src, dst], pe)
    pk, pi = _proj2t(ef_t, w_k, w_i, blk=1280)
    pa = _proj1t(af_t, w_a, blk=1280)
    msg = _act(cg_g, pstat_a, gb_atom, half=ad, residual=False, blk=1280)
    agg = _scatter_sc(msg, src, n)
    nodea = _nodeproj(agg[0], agg[1], vertex_feat, atom_out_w.T, w_j, blk=1000)

    g, pstat_g = _gsum_sc([nodea, pk, pi], [jj, kk, ii], pa)
    out_t = _act(g, pstat_g, gb_ang, half=gd, residual=True, blk=1280,
                 aft=af_t)
    return jnp.transpose(out_t, (1, 0))


# speedup vs baseline: 1.0590x; 1.0590x over previous
import functools

import jax
import jax.numpy as jnp
from jax import lax
from jax.experimental import pallas as pl
from jax.experimental.pallas import tpu as pltpu
from jax.experimental.pallas import tpu_sc as plsc

_NC, _NS, _LANES = 2, 16, 16
_NW = _NC * _NS
_CHUNK = 128


def _dot(a, b):
    return jnp.dot(a, b, preferred_element_type=jnp.float32)


def _proj2_body(x_ref, w1_ref, w2_ref, o1_ref, o2_ref):
    x = x_ref[...]
    o1_ref[...] = _dot(x, w1_ref[...])
    o2_ref[...] = _dot(x, w2_ref[...])


def _proj2(x, w1, w2, blk):
    n, d = x.shape
    return pl.pallas_call(
        _proj2_body,
        grid=(n // blk,),
        in_specs=[pl.BlockSpec((blk, d), lambda i: (i, 0)),
                  pl.BlockSpec(w1.shape, lambda i: (0, 0)),
                  pl.BlockSpec(w2.shape, lambda i: (0, 0))],
        out_specs=[pl.BlockSpec((blk, w1.shape[1]), lambda i: (i, 0)),
                   pl.BlockSpec((blk, w2.shape[1]), lambda i: (i, 0))],
        out_shape=[jax.ShapeDtypeStruct((n, w1.shape[1]), jnp.float32),
                   jax.ShapeDtypeStruct((n, w2.shape[1]), jnp.float32)],
    )(x, w1, w2)


def _dot_t(xt, w):
    return jax.lax.dot_general(xt, w, (((0,), (0,)), ((), ())),
                               preferred_element_type=jnp.float32)


def _proj3t_body(xt_ref, w1_ref, w2_ref, w3_ref, o1_ref, o2_ref, o3_ref):
    xt = xt_ref[...]
    o1_ref[...] = _dot_t(xt, w1_ref[...])
    o2_ref[...] = _dot_t(xt, w2_ref[...])
    o3_ref[...] = _dot_t(xt, w3_ref[...])


def _proj3t(xt, w1, w2, w3, blk):
    d, n = xt.shape
    return pl.pallas_call(
        _proj3t_body,
        grid=(n // blk,),
        in_specs=[pl.BlockSpec((d, blk), lambda i: (0, i)),
                  pl.BlockSpec(w1.shape, lambda i: (0, 0)),
                  pl.BlockSpec(w2.shape, lambda i: (0, 0)),
                  pl.BlockSpec(w3.shape, lambda i: (0, 0))],
        out_specs=[pl.BlockSpec((blk, w1.shape[1]), lambda i: (i, 0)),
                   pl.BlockSpec((blk, w2.shape[1]), lambda i: (i, 0)),
                   pl.BlockSpec((blk, w3.shape[1]), lambda i: (i, 0))],
        out_shape=[jax.ShapeDtypeStruct((n, w1.shape[1]), jnp.float32),
                   jax.ShapeDtypeStruct((n, w2.shape[1]), jnp.float32),
                   jax.ShapeDtypeStruct((n, w3.shape[1]), jnp.float32)],
    )(xt, w1, w2, w3)


def _proj1t_body(xt_ref, w_ref, o_ref):
    o_ref[...] = _dot_t(xt_ref[...], w_ref[...])


def _proj1t(xt, w, blk):
    d, n = xt.shape
    return pl.pallas_call(
        _proj1t_body,
        grid=(n // blk,),
        in_specs=[pl.BlockSpec((d, blk), lambda i: (0, i)),
                  pl.BlockSpec(w.shape, lambda i: (0, 0))],
        out_specs=pl.BlockSpec((blk, w.shape[1]), lambda i: (i, 0)),
        out_shape=jax.ShapeDtypeStruct((n, w.shape[1]), jnp.float32),
    )(xt, w)


def _sigmoid(x):
    return 0.5 * jnp.tanh(0.5 * x) + 0.5


def _act_body(n_rows, half, residual, *refs):
    if residual:
        xg_ref, sums_ref, sqs_ref, gb_ref, aft_ref, o_ref = refs
    else:
        xg_ref, sums_ref, sqs_ref, gb_ref, o_ref = refs
    cg = xg_ref[...]
    s = jnp.sum(sums_ref[...], axis=0, keepdims=True)
    q = jnp.sum(sqs_ref[...], axis=0, keepdims=True)
    inv_n = 1.0 / float(n_rows)
    mu = s * inv_n
    var = q * inv_n - mu * mu
    inv = lax.rsqrt(var + 1e-5)
    y = (cg - mu) * (inv * gb_ref[0:1, :]) + gb_ref[1:2, :]
    core = y[:, :half]
    gate = y[:, half:]
    x = core * _sigmoid(core) * _sigmoid(gate)
    if residual:
        o_ref[...] = jnp.transpose(x, (1, 0)) + aft_ref[...]
    else:
        o_ref[...] = x


def _act(xg, partials, gb, half, residual, blk, aft=None):
    n, wc = xg.shape
    ps = partials.reshape(_NW, 2, wc)
    sums = ps[:, 0, :]
    sqs = ps[:, 1, :]
    body = functools.partial(_act_body, n, half, residual)
    in_specs = [pl.BlockSpec((blk, wc), lambda i: (i, 0)),
                pl.BlockSpec((_NW, wc), lambda i: (0, 0)),
                pl.BlockSpec((_NW, wc), lambda i: (0, 0)),
                pl.BlockSpec((2, wc), lambda i: (0, 0))]
    args = [xg, sums, sqs, gb]
    if residual:
        in_specs.append(pl.BlockSpec((half, blk), lambda i: (0, i)))
        args.append(aft)
        out_specs = pl.BlockSpec((half, blk), lambda i: (0, i))
        out_shape = jax.ShapeDtypeStruct((half, n), jnp.float32)
    else:
        out_specs = pl.BlockSpec((blk, half), lambda i: (i, 0))
        out_shape = jax.ShapeDtypeStruct((n, half), jnp.float32)
    return pl.pallas_call(
        body,
        grid=(n // blk,),
        in_specs=in_specs,
        out_specs=out_specs,
        out_shape=out_shape,
    )(*args)


def _nodeproj_body(a0_ref, a1_ref, vf_ref, wo_ref, wj_ref, o_ref):
    nv = _dot(a0_ref[...] + a1_ref[...], wo_ref[...]) + vf_ref[...]
    o_ref[...] = _dot(nv, wj_ref[...])


def _nodeproj(a0, a1, vf, wo, wj, blk):
    n, d = vf.shape
    return pl.pallas_call(
        _nodeproj_body,
        grid=(n // blk,),
        in_specs=[pl.BlockSpec((blk, d), lambda i: (i, 0)),
                  pl.BlockSpec((blk, d), lambda i: (i, 0)),
                  pl.BlockSpec((blk, d), lambda i: (i, 0)),
                  pl.BlockSpec((d, d), lambda i: (0, 0)),
                  pl.BlockSpec((d, wj.shape[1]), lambda i: (0, 0))],
        out_specs=pl.BlockSpec((blk, wj.shape[1]), lambda i: (i, 0)),
        out_shape=jax.ShapeDtypeStruct((n, wj.shape[1]), jnp.float32),
    )(a0, a1, vf, wo, wj)


def _gsum_sc(tables, idxs, dense):
    nt = len(tables)
    m = idxs[0].shape[0]
    width = tables[0].shape[1]
    chunk = 64
    nseg = width // _LANES
    nchunk = m // chunk
    iters = (nchunk + _NW - 1) // _NW
    mesh = plsc.VectorSubcoreMesh(core_axis_name="c", subcore_axis_name="s")
    scratch = ([pltpu.VMEM((chunk,), jnp.int32) for _ in range(2 * nt)]
               + [pltpu.VMEM((chunk, width), jnp.float32)
                  for _ in range(2 * (nt + 1))]
               + [pltpu.VMEM((2, width), jnp.float32)]
               + [pltpu.SemaphoreType.DMA for _ in range(4)])

    def body(*refs):
        table_refs = refs[:nt]
        idx_refs = refs[nt:2 * nt]
        dense_ref = refs[2 * nt]
        out_ref = refs[2 * nt + 1]
        pstat_ref = refs[2 * nt + 2]
        sc = refs[2 * nt + 3:]
        ib = [sc[p * nt:(p + 1) * nt] for p in range(2)]
        bf = [sc[2 * nt + p * (nt + 1):2 * nt + (p + 1) * (nt + 1)]
              for p in range(2)]
        sacc = sc[4 * nt + 2]
        isem = sc[4 * nt + 3:4 * nt + 5]
        gsem = sc[4 * nt + 5:4 * nt + 7]
        cid = lax.axis_index("c")
        sid = lax.axis_index("s")
        wid = sid * _NC + cid
        zero = jnp.zeros((_LANES,), jnp.float32)

        for g in range(nseg):
            sacc[0, pl.ds(g * _LANES, _LANES)] = zero
            sacc[1, pl.ds(g * _LANES, _LANES)] = zero

        def load_idx(i, p):
            @pl.when(i * _NW + wid < nchunk)
            def _():
                base = (i * _NW + wid) * chunk
                for ti in range(nt):
                    pltpu.async_copy(idx_refs[ti].at[pl.ds(base, chunk)],
                                     ib[p][ti], isem[p])

        def wait_idx(i, p):
            @pl.when(i * _NW + wid < nchunk)
            def _():
                base = (i * _NW + wid) * chunk
                for ti in range(nt):
                    pltpu.make_async_copy(
                        idx_refs[ti].at[pl.ds(base, chunk)],
                        ib[p][ti], isem[p]).wait()

        def issue_gathers(i, p):
            @pl.when(i * _NW + wid < nchunk)
            def _():
                base = (i * _NW + wid) * chunk
                for ti in range(nt):
                    pltpu.async_copy(table_refs[ti].at[ib[p][ti]],
                                     bf[p][ti], gsem[p])
                pltpu.async_copy(dense_ref.at[pl.ds(base, chunk)],
                                 bf[p][nt], gsem[p])

        def wait_gathers(i, p):
            @pl.when(i * _NW + wid < nchunk)
            def _():
                base = (i * _NW + wid) * chunk
                for ti in range(nt):
                    pltpu.make_async_copy(table_refs[ti].at[ib[p][ti]],
                                          bf[p][ti], gsem[p]).wait()
                pltpu.make_async_copy(dense_ref.at[pl.ds(base, chunk)],
                                      bf[p][nt], gsem[p]).wait()

        def add_and_write(i, p):
            @pl.when(i * _NW + wid < nchunk)
            def _():
                def add_row(r, carry):
                    accs = list(carry)
                    for g in range(nseg):
                        sl = pl.ds(g * _LANES, _LANES)
                        x = bf[p][0][r, sl]
                        for ti in range(1, nt + 1):
                            x = x + bf[p][ti][r, sl]
                        bf[p][0][r, sl] = x
                        accs[2 * g] = accs[2 * g] + x
                        accs[2 * g + 1] = accs[2 * g + 1] + x * x
                    return tuple(accs)

                accs = lax.fori_loop(0, chunk, add_row,
                                     tuple(zero for _ in range(2 * nseg)))
                for g in range(nseg):
                    sl = pl.ds(g * _LANES, _LANES)
                    sacc[0, sl] = sacc[0, sl] + accs[2 * g]
                    sacc[1, sl] = sacc[1, sl] + accs[2 * g + 1]
                pltpu.sync_copy(
                    bf[p][0],
                    out_ref.at[pl.ds((i * _NW + wid) * chunk, chunk)])

        load_idx(0, 0)
        wait_idx(0, 0)
        issue_gathers(0, 0)

        def pair(ih, carry):
            for p in range(2):
                i = ih * 2 + p
                q = 1 - p
                load_idx(i + 1, q)
                wait_gathers(i, p)
                wait_idx(i + 1, q)
                issue_gathers(i + 1, q)
                add_and_write(i, p)
            return carry

        lax.fori_loop(0, (iters + 1) // 2, pair, 0)
        pltpu.sync_copy(sacc, pstat_ref.at[cid, sid])

    fn = pl.kernel(
        body,
        out_type=(jax.ShapeDtypeStruct((m, width), jnp.float32),
                  jax.ShapeDtypeStruct((_NC, _NS, 2, width), jnp.float32)),
        mesh=mesh,
        scratch_types=scratch,
        compiler_params=pltpu.CompilerParams(use_tc_tiling_on_sc=True),
    )
    return fn(*tables, *idxs, dense)


def _scatter_sc(msg, src, n):
    e, d = msg.shape
    nseg = d // _LANES
    nchunk = e // _CHUNK
    per_core = nchunk // _NC
    iters = (per_core + _NS - 1) // _NS
    npieces = (n + _CHUNK - 1) // _CHUNK
    plast = n - (npieces - 1) * _CHUNK
    piters = (npieces + _NS - 1) // _NS
    mesh = plsc.VectorSubcoreMesh(core_axis_name="c", subcore_axis_name="s")
    scratch = [pltpu.VMEM((_CHUNK, d), jnp.float32),
               pltpu.VMEM((_CHUNK, d), jnp.float32),
               pltpu.VMEM((_CHUNK,), jnp.int32),
               pltpu.VMEM((_CHUNK,), jnp.int32),
               pltpu.SemaphoreType.DMA,
               pltpu.SemaphoreType.DMA,
               pltpu.VMEM_SHARED((n, d), jnp.float32)]

    def body(msg_hbm, src_hbm, out_hbm, mbuf0, mbuf1, ibuf0, ibuf1,
             lsem0, lsem1, acc):
        mbuf = (mbuf0, mbuf1)
        ibuf = (ibuf0, ibuf1)
        lsem = (lsem0, lsem1)
        c = lax.axis_index("c")
        s = lax.axis_index("s")
        zero = jnp.zeros((_LANES,), jnp.float32)

        def zrow(r, cy):
            for g in range(nseg):
                mbuf0[r, pl.ds(g * _LANES, _LANES)] = zero
            return cy

        lax.fori_loop(0, _CHUNK, zrow, 0)

        def zpiece(i, cy):
            p = i * _NS + s

            @pl.when(p < npieces - 1)
            def _():
                pltpu.sync_copy(mbuf0, acc.at[pl.ds(p * _CHUNK, _CHUNK)])

            @pl.when(p == npieces - 1)
            def _():
                pltpu.sync_copy(mbuf0.at[pl.ds(0, plast)],
                                acc.at[pl.ds(p * _CHUNK, plast)])

            return cy

        lax.fori_loop(0, piters, zpiece, 0)
        plsc.subcore_barrier()

        def load(i, p):
            @pl.when(i * _NS + s < per_core)
            def _():
                base = (c * per_core + i * _NS + s) * _CHUNK
                pltpu.async_copy(src_hbm.at[pl.ds(base, _CHUNK)],
                                 ibuf[p], lsem[p])
                pltpu.async_copy(msg_hbm.at[pl.ds(base, _CHUNK)],
                                 mbuf[p], lsem[p])

        def wait_load(i, p):
            @pl.when(i * _NS + s < per_core)
            def _():
                base = (c * per_core + i * _NS + s) * _CHUNK
                pltpu.make_async_copy(src_hbm.at[pl.ds(base, _CHUNK)],
                                      ibuf[p], lsem[p]).wait()
                pltpu.make_async_copy(msg_hbm.at[pl.ds(base, _CHUNK)],
                                      mbuf[p], lsem[p]).wait()

        def scat(i, p):
            @pl.when(i * _NS + s < per_core)
            def _():
                pltpu.sync_copy(mbuf[p], acc.at[ibuf[p]], add=True)

        load(0, 0)

        def pair(ih, cy):
            for p in range(2):
                i = ih * 2 + p
                load(i + 1, 1 - p)
                wait_load(i, p)
                scat(i, p)
            return cy

        lax.fori_loop(0, (iters + 1) // 2, pair, 0)
        plsc.subcore_barrier()

        def wpiece(i, cy):
            p = i * _NS + s

            @pl.when(p < npieces - 1)
            def _():
                pltpu.sync_copy(acc.at[pl.ds(p * _CHUNK, _CHUNK)],
                                out_hbm.at[c, pl.ds(p * _CHUNK, _CHUNK)])

            @pl.when(p == npieces - 1)
            def _():
                pltpu.sync_copy(acc.at[pl.ds(p * _CHUNK, plast)],
                                out_hbm.at[c, pl.ds(p * _CHUNK, plast)])

            return cy

        lax.fori_loop(0, piters, wpiece, 0)

    fn = pl.kernel(
        body,
        out_type=jax.ShapeDtypeStruct((_NC, n, d), jnp.float32),
        mesh=mesh,
        scratch_types=scratch,
        compiler_params=pltpu.CompilerParams(use_tc_tiling_on_sc=True),
    )
    return fn(msg, src)


def kernel(vertex_feat, edge_feat, angle_feat, atom_core_w, atom_gate_w,
           atom_bn_core_g, atom_bn_core_b, atom_bn_gate_g, atom_bn_gate_b,
           atom_out_w, angle_core_w, angle_gate_w, angle_bn_core_g,
           angle_bn_core_b, angle_bn_gate_g, angle_bn_gate_b,
           edge_index, k_idx, j_idx, i_idx):
    n, ad = vertex_feat.shape
    e, bd = edge_feat.shape
    t, gd = angle_feat.shape

    src = edge_index[0].astype(jnp.int32)
    dst = edge_index[1].astype(jnp.int32)
    kk = k_idx.astype(jnp.int32)
    jj = j_idx.astype(jnp.int32)
    ii = i_idx.astype(jnp.int32)

    w_src = jnp.concatenate([atom_core_w[:, :ad].T, atom_gate_w[:, :ad].T], 1)
    w_dst = jnp.concatenate([atom_core_w[:, ad + bd:].T,
                             atom_gate_w[:, ad + bd:].T], 1)
    w_e = jnp.concatenate([atom_core_w[:, ad:ad + bd].T,
                           atom_gate_w[:, ad:ad + bd].T], 1)
    gb_atom = jnp.stack([jnp.concatenate([atom_bn_core_g, atom_bn_gate_g]),
                         jnp.concatenate([atom_bn_core_b, atom_bn_gate_b])])
    w_j = jnp.concatenate([angle_core_w[:, :ad].T, angle_gate_w[:, :ad].T], 1)
    w_k = jnp.concatenate([angle_core_w[:, ad:ad + bd].T,
                           angle_gate_w[:, ad:ad + bd].T], 1)
    w_i = jnp.concatenate([angle_core_w[:, ad + bd:ad + 2 * bd].T,
                           angle_gate_w[:, ad + bd:ad + 2 * bd].T], 1)
    w_a = jnp.concatenate([angle_core_w[:, ad + 2 * bd:].T,
                           angle_gate_w[:, ad + 2 * bd:].T], 1)
    gb_ang = jnp.stack([jnp.concatenate([angle_bn_core_g, angle_bn_gate_g]),
                        jnp.concatenate([angle_bn_core_b, angle_bn_gate_b])])

    ef_t = jnp.transpose(edge_feat, (1, 0))
    af_t = jnp.transpose(angle_feat, (1, 0))

    t_src, t_dst = _proj2(vertex_feat, w_src, w_dst, blk=1000)
    pe, pk, pi = _proj3t(ef_t, w_e, w_k, w_i, blk=1280)
    pa = _proj1t(af_t, w_a, blk=1280)
    cg_g, pstat_a = _gsum_sc([t_src, t_dst], [src, dst], pe)
    msg = _act(cg_g, pstat_a, gb_atom, half=ad, residual=False, blk=1280)
    agg = _scatter_sc(msg, src, n)
    nodea = _nodeproj(agg[0], agg[1], vertex_feat, atom_out_w.T, w_j, blk=1000)

    g, pstat_g = _gsum_sc([nodea, pk, pi], [jj, kk, ii], pa)
    out_t = _act(g, pstat_g, gb_ang, half=gd, residual=True, blk=1280,
                 aft=af_t)
    return jnp.transpose(out_t, (1, 0))
